# speedup vs baseline: 19.2578x
import jax
import jax.numpy as jnp
from jax.experimental import pallas as pl


def kernel(node_type, edge_index, table):
    raise NotImplementedError("write your pallas kernel here")


# speedup vs baseline: 18.7522x; 18.7522x over previous
import functools

import jax
import jax.numpy as jnp
from jax import lax
from jax.experimental import pallas as pl
from jax.experimental.pallas import tpu as pltpu
from jax.experimental.pallas import tpu_sc as plsc

DIM = 128
N_NODES = 10000
N_EDGES = 320000

NC = 2
NS = 16
NW = NC * NS
B_PER_W = N_EDGES // NW
CHUNK = 80
N_CHUNKS = B_PER_W // CHUNK
LANES = 16
N_GROUPS_C = B_PER_W // LANES
NBUF = 5
GROUPS = N_CHUNKS // NBUF


def _edge_embed(node_type, edge_index, table):
    mesh = plsc.VectorSubcoreMesh(core_axis_name="c", subcore_axis_name="s")

    @functools.partial(
        pl.kernel,
        mesh=mesh,
        out_type=jax.ShapeDtypeStruct((N_EDGES, DIM), jnp.float32),
        scratch_types=[
            pltpu.VMEM((N_NODES,), jnp.int32),
            pltpu.VMEM((B_PER_W,), jnp.int32),
            pltpu.VMEM((B_PER_W,), jnp.int32),
            pltpu.VMEM((B_PER_W,), jnp.int32),
            pltpu.VMEM((NBUF, CHUNK, DIM), jnp.float32),
            pltpu.SemaphoreType.DMA((NBUF,)),
            pltpu.SemaphoreType.DMA((NBUF,)),
        ],
        compiler_params=pltpu.CompilerParams(needs_layout_passes=False),
    )
    def k(nt_hbm, src_hbm, dst_hbm, tbl_hbm, out_hbm, nt_v, src_v, dst_v, et_v, rows_v, gsem, osem):
        wid = lax.axis_index("s") * NC + lax.axis_index("c")
        base = wid * B_PER_W

        pltpu.sync_copy(nt_hbm, nt_v)
        pltpu.sync_copy(src_hbm.at[pl.ds(base, B_PER_W)], src_v)
        pltpu.sync_copy(dst_hbm.at[pl.ds(base, B_PER_W)], dst_v)

        def comp(i, _):
            off = i * LANES
            s16 = src_v[pl.ds(off, LANES)]
            d16 = dst_v[pl.ds(off, LANES)]
            a = plsc.load_gather(nt_v, [s16])
            b = plsc.load_gather(nt_v, [d16])
            ssum = a + b
            et_v[pl.ds(off, LANES)] = jnp.right_shift(ssum * (ssum + 1), 1) + b
            return 0

        lax.fori_loop(0, N_GROUPS_C, comp, 0)

        def do_group(j, _):
            for b in range(NBUF):
                c = j * NBUF + b

                @pl.when(c >= NBUF)
                def _():
                    pltpu.make_async_copy(
                        rows_v.at[b], out_hbm.at[pl.ds(0, CHUNK)], osem.at[b]
                    ).wait()

                pltpu.async_copy(
                    tbl_hbm.at[et_v.at[pl.ds(c * CHUNK, CHUNK)]],
                    rows_v.at[b],
                    gsem.at[b],
                )

            for b in range(NBUF):
                c = j * NBUF + b
                pltpu.make_async_copy(
                    tbl_hbm.at[et_v.at[pl.ds(0, CHUNK)]], rows_v.at[b], gsem.at[b]
                ).wait()
                pltpu.async_copy(
                    rows_v.at[b], out_hbm.at[pl.ds(base + c * CHUNK, CHUNK)], osem.at[b]
                )

            return 0

        lax.fori_loop(0, GROUPS, do_group, 0)

        for b in range(NBUF):
            pltpu.make_async_copy(
                rows_v.at[b], out_hbm.at[pl.ds(0, CHUNK)], osem.at[b]
            ).wait()

    return k


def kernel(node_type, edge_index, table):
    table = table.at[0].set(0.0)
    src = edge_index[0]
    dst = edge_index[1]
    return _edge_embed(node_type, edge_index, table)(node_type, src, dst, table)


# speedup vs baseline: 45.6367x; 2.4337x over previous
import functools

import jax
import jax.numpy as jnp
from jax import lax
from jax.experimental import pallas as pl
from jax.experimental.pallas import tpu as pltpu
from jax.experimental.pallas import tpu_sc as plsc

DIM = 128
N_NODES = 10000
N_EDGES = 320000
N_ROWS = 3000

NC = 2
NS = 16
NW = NC * NS
B_PER_W = N_EDGES // NW
CHUNK = 80
N_CHUNKS = B_PER_W // CHUNK
LANES = 16
NBUF = 4
GROUPS = (N_CHUNKS + NBUF - 1) // NBUF
STAGE = N_ROWS // NS


def _edge_embed(node_type, edge_index, table):
    mesh = plsc.VectorSubcoreMesh(core_axis_name="c", subcore_axis_name="s")

    @functools.partial(
        pl.kernel,
        mesh=mesh,
        out_type=jax.ShapeDtypeStruct((N_EDGES, DIM), jnp.float32),
        scratch_types=[
            pltpu.VMEM((N_NODES,), jnp.int32),
            pltpu.VMEM((B_PER_W,), jnp.int32),
            pltpu.VMEM((B_PER_W,), jnp.int32),
            pltpu.VMEM((B_PER_W,), jnp.int32),
            pltpu.VMEM((NBUF, CHUNK, DIM), jnp.float32),
            pltpu.VMEM_SHARED((N_ROWS, DIM), jnp.float32),
            pltpu.SemaphoreType.DMA((NBUF,)),
            pltpu.SemaphoreType.DMA((NBUF,)),
        ],
        compiler_params=pltpu.CompilerParams(needs_layout_passes=False),
    )
    def k(nt_hbm, src_hbm, dst_hbm, tbl_hbm, out_hbm,
          nt_v, src_v, dst_v, et_v, rows_v, tbl_sh, gsem, osem):
        sid = lax.axis_index("s")
        wid = sid * NC + lax.axis_index("c")
        base = wid * B_PER_W

        @pl.when(sid < 15)
        def _():
            pltpu.sync_copy(
                tbl_hbm.at[pl.ds(sid * 200, 200)], tbl_sh.at[pl.ds(sid * 200, 200)]
            )

        pltpu.sync_copy(nt_hbm, nt_v)
        pltpu.sync_copy(src_hbm.at[pl.ds(base, B_PER_W)], src_v)
        pltpu.sync_copy(dst_hbm.at[pl.ds(base, B_PER_W)], dst_v)
        plsc.subcore_barrier()

        def do_group(j, _):
            for b in range(NBUF):
                c = j * NBUF + b

                @pl.when(c < N_CHUNKS)
                def _():
                    for i in range(CHUNK // LANES):
                        off = c * CHUNK + i * LANES
                        s16 = src_v[pl.ds(off, LANES)]
                        d16 = dst_v[pl.ds(off, LANES)]
                        a = plsc.load_gather(nt_v, [s16])
                        bt = plsc.load_gather(nt_v, [d16])
                        ssum = a + bt
                        et_v[pl.ds(off, LANES)] = (
                            jnp.right_shift(ssum * (ssum + 1), 1) + bt
                        )

                    @pl.when(c >= NBUF)
                    def _():
                        pltpu.make_async_copy(
                            rows_v.at[b], out_hbm.at[pl.ds(0, CHUNK)], osem.at[b]
                        ).wait()

                    pltpu.async_copy(
                        tbl_sh.at[et_v.at[pl.ds(c * CHUNK, CHUNK)]],
                        rows_v.at[b],
                        gsem.at[b],
                    )

            for b in range(NBUF):
                c = j * NBUF + b

                @pl.when(c < N_CHUNKS)
                def _():
                    pltpu.make_async_copy(
                        tbl_sh.at[et_v.at[pl.ds(0, CHUNK)]], rows_v.at[b], gsem.at[b]
                    ).wait()
                    pltpu.async_copy(
                        rows_v.at[b], out_hbm.at[pl.ds(base + c * CHUNK, CHUNK)], osem.at[b]
                    )

            return 0

        lax.fori_loop(0, GROUPS, do_group, 0)

        for b in range(NBUF):
            pltpu.make_async_copy(
                rows_v.at[b], out_hbm.at[pl.ds(0, CHUNK)], osem.at[b]
            ).wait()

    return k


def kernel(node_type, edge_index, table):
    table = table.at[0].set(0.0)
    src = edge_index[0]
    dst = edge_index[1]
    return _edge_embed(node_type, edge_index, table)(node_type, src, dst, table)


# speedup vs baseline: 45.8775x; 1.0053x over previous
import functools

import jax
import jax.numpy as jnp
from jax import lax
from jax.experimental import pallas as pl
from jax.experimental.pallas import tpu as pltpu
from jax.experimental.pallas import tpu_sc as plsc

DIM = 128
N_NODES = 10000
N_EDGES = 320000
N_ROWS = 3000

NC = 2
NS = 16
NW = NC * NS
B_PER_W = N_EDGES // NW
CHUNK = 80
N_CHUNKS = B_PER_W // CHUNK
LANES = 16
NBUF = 6
GROUPS = (N_CHUNKS + NBUF - 1) // NBUF
STAGE = N_ROWS // NS


def _edge_embed(node_type, edge_index, table):
    mesh = plsc.VectorSubcoreMesh(core_axis_name="c", subcore_axis_name="s")

    @functools.partial(
        pl.kernel,
        mesh=mesh,
        out_type=jax.ShapeDtypeStruct((N_EDGES, DIM), jnp.float32),
        scratch_types=[
            pltpu.VMEM((N_NODES,), jnp.int32),
            pltpu.VMEM((B_PER_W,), jnp.int32),
            pltpu.VMEM((B_PER_W,), jnp.int32),
            pltpu.VMEM((B_PER_W,), jnp.int32),
            pltpu.VMEM((NBUF, CHUNK, DIM), jnp.float32),
            pltpu.VMEM_SHARED((N_ROWS, DIM), jnp.float32),
            pltpu.SemaphoreType.DMA((NBUF,)),
            pltpu.SemaphoreType.DMA((NBUF,)),
        ],
        compiler_params=pltpu.CompilerParams(needs_layout_passes=False),
    )
    def k(nt_hbm, src_hbm, dst_hbm, tbl_hbm, out_hbm,
          nt_v, src_v, dst_v, et_v, rows_v, tbl_sh, gsem, osem):
        sid = lax.axis_index("s")
        wid = sid * NC + lax.axis_index("c")
        base = wid * B_PER_W

        @pl.when(sid < 15)
        def _():
            pltpu.sync_copy(
                tbl_hbm.at[pl.ds(sid * 200, 200)], tbl_sh.at[pl.ds(sid * 200, 200)]
            )

        pltpu.sync_copy(nt_hbm, nt_v)
        pltpu.sync_copy(src_hbm.at[pl.ds(base, B_PER_W)], src_v)
        pltpu.sync_copy(dst_hbm.at[pl.ds(base, B_PER_W)], dst_v)
        plsc.subcore_barrier()

        def do_group(j, _):
            for b in range(NBUF):
                c = j * NBUF + b

                @pl.when(c < N_CHUNKS)
                def _():
                    for i in range(CHUNK // LANES):
                        off = c * CHUNK + i * LANES
                        s16 = src_v[pl.ds(off, LANES)]
                        d16 = dst_v[pl.ds(off, LANES)]
                        a = plsc.load_gather(nt_v, [s16])
                        bt = plsc.load_gather(nt_v, [d16])
                        ssum = a + bt
                        et_v[pl.ds(off, LANES)] = (
                            jnp.right_shift(ssum * (ssum + 1), 1) + bt
                        )

                    @pl.when(c >= NBUF)
                    def _():
                        pltpu.make_async_copy(
                            rows_v.at[b], out_hbm.at[pl.ds(0, CHUNK)], osem.at[b]
                        ).wait()

                    pltpu.async_copy(
                        tbl_sh.at[et_v.at[pl.ds(c * CHUNK, CHUNK)]],
                        rows_v.at[b],
                        gsem.at[b],
                    )

            for b in range(NBUF):
                c = j * NBUF + b

                @pl.when(c < N_CHUNKS)
                def _():
                    pltpu.make_async_copy(
                        tbl_sh.at[et_v.at[pl.ds(0, CHUNK)]], rows_v.at[b], gsem.at[b]
                    ).wait()
                    pltpu.async_copy(
                        rows_v.at[b], out_hbm.at[pl.ds(base + c * CHUNK, CHUNK)], osem.at[b]
                    )

            return 0

        lax.fori_loop(0, GROUPS, do_group, 0)

        for b in range(NBUF):
            pltpu.make_async_copy(
                rows_v.at[b], out_hbm.at[pl.ds(0, CHUNK)], osem.at[b]
            ).wait()

    return k


def kernel(node_type, edge_index, table):
    table = table.at[0].set(0.0)
    src = edge_index[0]
    dst = edge_index[1]
    return _edge_embed(node_type, edge_index, table)(node_type, src, dst, table)


# speedup vs baseline: 46.4777x; 1.0131x over previous
import functools

import jax
import jax.numpy as jnp
from jax import lax
from jax.experimental import pallas as pl
from jax.experimental.pallas import tpu as pltpu
from jax.experimental.pallas import tpu_sc as plsc

DIM = 128
N_NODES = 10000
N_EDGES = 320000
N_ROWS = 3000

NC = 2
NS = 16
NW = NC * NS
B_PER_W = N_EDGES // NW
CHUNK = 80
N_CHUNKS = B_PER_W // CHUNK
LANES = 16
NBUF = 6
GROUPS = (N_CHUNKS + NBUF - 1) // NBUF
STAGE = N_ROWS // NS


def _edge_embed(node_type, edge_index, table):
    mesh = plsc.VectorSubcoreMesh(core_axis_name="c", subcore_axis_name="s")

    @functools.partial(
        pl.kernel,
        mesh=mesh,
        out_type=jax.ShapeDtypeStruct((N_EDGES, DIM), jnp.float32),
        scratch_types=[
            pltpu.VMEM((N_NODES,), jnp.int32),
            pltpu.VMEM((B_PER_W,), jnp.int32),
            pltpu.VMEM((B_PER_W,), jnp.int32),
            pltpu.VMEM((B_PER_W,), jnp.int32),
            pltpu.VMEM((NBUF, CHUNK, DIM), jnp.float32),
            pltpu.VMEM_SHARED((N_ROWS, DIM), jnp.float32),
            pltpu.SemaphoreType.DMA((NBUF,)),
            pltpu.SemaphoreType.DMA((NBUF,)),
            pltpu.SemaphoreType.DMA((4,)),
        ],
        compiler_params=pltpu.CompilerParams(needs_layout_passes=False),
    )
    def k(nt_hbm, src_hbm, dst_hbm, tbl_hbm, out_hbm,
          nt_v, src_v, dst_v, et_v, rows_v, tbl_sh, gsem, osem, isem):
        sid = lax.axis_index("s")
        wid = sid * NC + lax.axis_index("c")
        base = wid * B_PER_W

        @pl.when(sid < 15)
        def _():
            pltpu.async_copy(
                tbl_hbm.at[pl.ds(sid * 200, 200)],
                tbl_sh.at[pl.ds(sid * 200, 200)],
                isem.at[0],
            )

        pltpu.async_copy(nt_hbm, nt_v, isem.at[1])
        pltpu.async_copy(src_hbm.at[pl.ds(base, B_PER_W)], src_v, isem.at[2])
        pltpu.async_copy(dst_hbm.at[pl.ds(base, B_PER_W)], dst_v, isem.at[3])

        @pl.when(sid < 15)
        def _():
            pltpu.make_async_copy(
                tbl_hbm.at[pl.ds(sid * 200, 200)],
                tbl_sh.at[pl.ds(sid * 200, 200)],
                isem.at[0],
            ).wait()

        pltpu.make_async_copy(nt_hbm, nt_v, isem.at[1]).wait()
        pltpu.make_async_copy(src_hbm.at[pl.ds(base, B_PER_W)], src_v, isem.at[2]).wait()
        pltpu.make_async_copy(dst_hbm.at[pl.ds(base, B_PER_W)], dst_v, isem.at[3]).wait()
        plsc.subcore_barrier()

        def do_group(j, _):
            for b in range(NBUF):
                c = j * NBUF + b

                @pl.when(c < N_CHUNKS)
                def _():
                    for i in range(CHUNK // LANES):
                        off = c * CHUNK + i * LANES
                        s16 = src_v[pl.ds(off, LANES)]
                        d16 = dst_v[pl.ds(off, LANES)]
                        a = plsc.load_gather(nt_v, [s16])
                        bt = plsc.load_gather(nt_v, [d16])
                        ssum = a + bt
                        et_v[pl.ds(off, LANES)] = (
                            jnp.right_shift(ssum * (ssum + 1), 1) + bt
                        )

                    @pl.when(c >= NBUF)
                    def _():
                        pltpu.make_async_copy(
                            rows_v.at[b], out_hbm.at[pl.ds(0, CHUNK)], osem.at[b]
                        ).wait()

                    pltpu.async_copy(
                        tbl_sh.at[et_v.at[pl.ds(c * CHUNK, CHUNK)]],
                        rows_v.at[b],
                        gsem.at[b],
                    )

            for b in range(NBUF):
                c = j * NBUF + b

                @pl.when(c < N_CHUNKS)
                def _():
                    pltpu.make_async_copy(
                        tbl_sh.at[et_v.at[pl.ds(0, CHUNK)]], rows_v.at[b], gsem.at[b]
                    ).wait()
                    pltpu.async_copy(
                        rows_v.at[b], out_hbm.at[pl.ds(base + c * CHUNK, CHUNK)], osem.at[b]
                    )

            return 0

        lax.fori_loop(0, GROUPS, do_group, 0)

        for b in range(NBUF):
            pltpu.make_async_copy(
                rows_v.at[b], out_hbm.at[pl.ds(0, CHUNK)], osem.at[b]
            ).wait()

    return k


def kernel(node_type, edge_index, table):
    table = table.at[0].set(0.0)
    src = edge_index[0]
    dst = edge_index[1]
    return _edge_embed(node_type, edge_index, table)(node_type, src, dst, table)
